# Initial kernel scaffold; baseline (speedup 1.0000x reference)
import jax
import jax.numpy as jnp
from jax.experimental import pallas as pl


def kernel(prediction):
    raise NotImplementedError("write your pallas kernel here")


# speedup vs baseline: 1.1721x; 1.1721x over previous
import functools

import jax
import jax.numpy as jnp
import numpy as np
from jax import lax
from jax.experimental import pallas as pl
from jax.experimental.pallas import tpu as pltpu
from jax.experimental.pallas import tpu_sc as plsc

_B, _H, _W, _C = 4, 512, 512, 32
_HW = _H * _W
_K = 1000
_KPAD = 1024
_NBIN = 12
_THRESH = 0.2


def _mask_kernel(p0_ref, p1_ref, out_ref, scratch):
    p0 = p0_ref[0]
    p1 = p1_ref[0]
    m = jnp.maximum(p0, p1)
    e0 = jnp.exp(p0 - m)
    e1 = jnp.exp(p1 - m)
    h = e1 / (e0 + e1)
    scratch[...] = jnp.full((528, 768), -jnp.inf, jnp.float32)
    scratch[pl.ds(8, _H), pl.ds(128, _W)] = h
    pool = jnp.full((_H, _W), -jnp.inf, jnp.float32)
    for di in (7, 8, 9):
        for dj in (127, 128, 129):
            pool = jnp.maximum(pool, scratch[pl.ds(di, _H), pl.ds(dj, _W)])
    keep = jnp.logical_and(h > _THRESH, h == pool)
    out_ref[0] = jnp.where(keep, h, 0.0)


def _mask_call(p0, p1):
    return pl.pallas_call(
        _mask_kernel,
        grid=(_B,),
        in_specs=[
            pl.BlockSpec((1, _H, _W), lambda b: (b, 0, 0)),
            pl.BlockSpec((1, _H, _W), lambda b: (b, 0, 0)),
        ],
        out_specs=pl.BlockSpec((1, _H, _W), lambda b: (b, 0, 0)),
        out_shape=jax.ShapeDtypeStruct((_B, _H, _W), jnp.float32),
        scratch_shapes=[pltpu.VMEM((528, 768), jnp.float32)],
        interpret=False,
    )(p0, p1)


def _decode_kernel(rows_ref, idx_ref, out_ref):
    n = _B * _KPAD
    rows128 = rows_ref[...]
    idxf = idx_ref[...]
    sub = idxf - jnp.floor(idxf * 0.25) * 4.0
    rows = jnp.zeros((n, _C), jnp.float32)
    for m in range(4):
        rows = rows + jnp.where(sub == m, rows128[:, m * _C:(m + 1) * _C], 0.0)
    col = lax.broadcasted_iota(jnp.int32, (n, _C), 1)
    colf = col.astype(jnp.float32)
    angle = 2.0 * np.pi / _NBIN

    def sel(c):
        return jnp.sum(jnp.where(col == c, rows, 0.0), axis=1, keepdims=True)

    binm = jnp.logical_and(col >= 5, col <= 16)
    bmax = jnp.max(jnp.where(binm, rows, -jnp.inf), axis=1, keepdims=True)
    bidx = jnp.min(
        jnp.where(jnp.logical_and(binm, rows == bmax), colf - 5.0, 1e9),
        axis=1, keepdims=True)
    res = jnp.sum(
        jnp.where(jnp.logical_and(col >= 17, colf - 17.0 == bidx), rows, 0.0),
        axis=1, keepdims=True)
    heading = jnp.mod(bidx * angle + res * (angle * 0.5), 2.0 * np.pi)
    heading = jnp.where(heading > np.pi, heading - 2.0 * np.pi, heading)

    sl = sel(29) * 4.7 + 4.7
    sw = sel(30) * 2.1 + 2.1
    sh = sel(31) * 1.7 + 1.7
    ix = jnp.floor(idxf / 512.0)
    iy = idxf - ix * 512.0
    cx = (-81.92 + (ix + 0.5) * 0.32) + sel(2)
    cy = (-81.92 + (iy + 0.5) * 0.32) + sel(3)
    cz = sel(4)

    ocol = lax.broadcasted_iota(jnp.int32, (n, 8), 1)
    out = jnp.zeros((n, 8), jnp.float32)
    for c, v in enumerate((cx, cy, cz, sl, sw, sh, heading)):
        out = out + jnp.where(ocol == c, v, 0.0)
    out_ref[...] = out


def _decode_call(rows, idxf):
    return pl.pallas_call(
        _decode_kernel,
        out_shape=jax.ShapeDtypeStruct((_B * _KPAD, 8), jnp.float32),
        interpret=False,
    )(rows, idxf)


def _sc_gather(table, gidx):
    info = plsc.get_sparse_core_info()
    nc, ns = info.num_cores, info.num_subcores
    nw = nc * ns
    bpw = (_B * _KPAD) // nw
    mesh = plsc.VectorSubcoreMesh(core_axis_name="c", subcore_axis_name="s")

    @functools.partial(
        pl.kernel, mesh=mesh,
        out_type=jax.ShapeDtypeStruct((_B * _KPAD, 128), jnp.float32),
        scratch_types=[
            pltpu.VMEM((bpw,), jnp.int32),
            pltpu.VMEM((bpw, 128), jnp.float32),
            pltpu.SemaphoreType.DMA,
        ],
    )
    def gk(table_hbm, idx_hbm, out_hbm, idx_v, rows_v, sem):
        wid = lax.axis_index("s") * nc + lax.axis_index("c")
        base = wid * bpw
        pltpu.sync_copy(idx_hbm.at[pl.ds(base, bpw)], idx_v)
        pltpu.async_copy(table_hbm.at[idx_v], rows_v, sem).wait()
        pltpu.sync_copy(rows_v, out_hbm.at[pl.ds(base, bpw)])

    return gk(table, gidx)


def kernel(prediction):
    p0 = prediction[..., 0]
    p1 = prediction[..., 1]
    s = _mask_call(p0, p1)
    vals, top_idx = lax.top_k(s.reshape(_B, _HW), _K)
    pad_idx = jnp.pad(top_idx, ((0, 0), (0, _KPAD - _K)))
    gidx = (pad_idx
            + (jnp.arange(_B, dtype=jnp.int32) * _HW)[:, None]).reshape(-1)
    gsup = gidx // 4
    table = prediction.reshape((_B * _HW) // 4, _C * 4)
    rows = _sc_gather(table, gsup)
    idxf = pad_idx.astype(jnp.float32).reshape(_B * _KPAD, 1)
    boxes = _decode_call(rows, idxf)
    box_decoded = boxes.reshape(_B, _KPAD, 8)[:, :_K, :7]
    box_class = jnp.full((_B, _K), 1, jnp.int32)
    return box_decoded, box_class, vals


# speedup vs baseline: 1.1757x; 1.0030x over previous
import functools

import jax
import jax.numpy as jnp
import numpy as np
from jax import lax
from jax.experimental import pallas as pl
from jax.experimental.pallas import tpu as pltpu
from jax.experimental.pallas import tpu_sc as plsc

_B, _H, _W, _C = 4, 512, 512, 32
_HW = _H * _W
_K = 1000
_KPAD = 1024
_NBIN = 12
_THRESH = 0.2


def _mask_kernel(d_ref, out_ref, scratch):
    d = d_ref[0]
    ed = jnp.exp(-jnp.abs(d))
    h = jnp.where(d >= 0.0, 1.0 / (ed + 1.0), ed / (1.0 + ed))
    scratch[...] = jnp.full((528, 768), -jnp.inf, jnp.float32)
    scratch[pl.ds(8, _H), pl.ds(128, _W)] = h
    pool = jnp.full((_H, _W), -jnp.inf, jnp.float32)
    for di in (7, 8, 9):
        for dj in (127, 128, 129):
            pool = jnp.maximum(pool, scratch[pl.ds(di, _H), pl.ds(dj, _W)])
    keep = jnp.logical_and(h > _THRESH, h == pool)
    out_ref[0] = jnp.where(keep, h, 0.0)


def _mask_call(d):
    return pl.pallas_call(
        _mask_kernel,
        grid=(_B,),
        in_specs=[
            pl.BlockSpec((1, _H, _W), lambda b: (b, 0, 0)),
        ],
        out_specs=pl.BlockSpec((1, _H, _W), lambda b: (b, 0, 0)),
        out_shape=jax.ShapeDtypeStruct((_B, _H, _W), jnp.float32),
        scratch_shapes=[pltpu.VMEM((528, 768), jnp.float32)],
        interpret=False,
    )(d)


def _decode_kernel(rows_ref, idx_ref, out_ref):
    n = _B * _KPAD
    rows128 = rows_ref[...]
    idxf = idx_ref[...]
    sub = idxf - jnp.floor(idxf * 0.25) * 4.0
    rows = jnp.zeros((n, _C), jnp.float32)
    for m in range(4):
        rows = rows + jnp.where(sub == m, rows128[:, m * _C:(m + 1) * _C], 0.0)
    col = lax.broadcasted_iota(jnp.int32, (n, _C), 1)
    colf = col.astype(jnp.float32)
    angle = 2.0 * np.pi / _NBIN

    def sel(c):
        return jnp.sum(jnp.where(col == c, rows, 0.0), axis=1, keepdims=True)

    binm = jnp.logical_and(col >= 5, col <= 16)
    bmax = jnp.max(jnp.where(binm, rows, -jnp.inf), axis=1, keepdims=True)
    bidx = jnp.min(
        jnp.where(jnp.logical_and(binm, rows == bmax), colf - 5.0, 1e9),
        axis=1, keepdims=True)
    res = jnp.sum(
        jnp.where(jnp.logical_and(col >= 17, colf - 17.0 == bidx), rows, 0.0),
        axis=1, keepdims=True)
    heading = jnp.mod(bidx * angle + res * (angle * 0.5), 2.0 * np.pi)
    heading = jnp.where(heading > np.pi, heading - 2.0 * np.pi, heading)

    sl = sel(29) * 4.7 + 4.7
    sw = sel(30) * 2.1 + 2.1
    sh = sel(31) * 1.7 + 1.7
    ix = jnp.floor(idxf / 512.0)
    iy = idxf - ix * 512.0
    cx = (-81.92 + (ix + 0.5) * 0.32) + sel(2)
    cy = (-81.92 + (iy + 0.5) * 0.32) + sel(3)
    cz = sel(4)

    ocol = lax.broadcasted_iota(jnp.int32, (n, 8), 1)
    out = jnp.zeros((n, 8), jnp.float32)
    for c, v in enumerate((cx, cy, cz, sl, sw, sh, heading)):
        out = out + jnp.where(ocol == c, v, 0.0)
    out_ref[...] = out


def _decode_call(rows, idxf):
    return pl.pallas_call(
        _decode_kernel,
        out_shape=jax.ShapeDtypeStruct((_B * _KPAD, 8), jnp.float32),
        interpret=False,
    )(rows, idxf)


def _sc_gather(table, gidx):
    info = plsc.get_sparse_core_info()
    nc, ns = info.num_cores, info.num_subcores
    nw = nc * ns
    bpw = (_B * _KPAD) // nw
    mesh = plsc.VectorSubcoreMesh(core_axis_name="c", subcore_axis_name="s")

    @functools.partial(
        pl.kernel, mesh=mesh,
        out_type=jax.ShapeDtypeStruct((_B * _KPAD, 128), jnp.float32),
        scratch_types=[
            pltpu.VMEM((bpw,), jnp.int32),
            pltpu.VMEM((bpw, 128), jnp.float32),
            pltpu.SemaphoreType.DMA,
        ],
    )
    def gk(table_hbm, idx_hbm, out_hbm, idx_v, rows_v, sem):
        wid = lax.axis_index("s") * nc + lax.axis_index("c")
        base = wid * bpw
        pltpu.sync_copy(idx_hbm.at[pl.ds(base, bpw)], idx_v)
        pltpu.async_copy(table_hbm.at[idx_v], rows_v, sem).wait()
        pltpu.sync_copy(rows_v, out_hbm.at[pl.ds(base, bpw)])

    return gk(table, gidx)


def kernel(prediction):
    d = prediction[..., 1] - prediction[..., 0]
    s = _mask_call(d)
    vals, top_idx = lax.top_k(s.reshape(_B, _HW), _K)
    pad_idx = jnp.pad(top_idx, ((0, 0), (0, _KPAD - _K)))
    gidx = (pad_idx
            + (jnp.arange(_B, dtype=jnp.int32) * _HW)[:, None]).reshape(-1)
    gsup = gidx // 4
    table = prediction.reshape((_B * _HW) // 4, _C * 4)
    rows = _sc_gather(table, gsup)
    idxf = pad_idx.astype(jnp.float32).reshape(_B * _KPAD, 1)
    boxes = _decode_call(rows, idxf)
    box_decoded = boxes.reshape(_B, _KPAD, 8)[:, :_K, :7]
    box_class = jnp.full((_B, _K), 1, jnp.int32)
    return box_decoded, box_class, vals


# speedup vs baseline: 2.0983x; 1.7848x over previous
import functools

import jax
import jax.numpy as jnp
import numpy as np
from jax import lax
from jax.experimental import pallas as pl
from jax.experimental.pallas import tpu as pltpu
from jax.experimental.pallas import tpu_sc as plsc

_B, _H, _W, _C = 4, 512, 512, 32
_HW = _H * _W
_K = 1000
_KPAD = 1024
_NBIN = 12
_THRESH = 0.2


def _mask_kernel(d_ref, out_ref, scratch):
    d = d_ref[0]
    ed = jnp.exp(-jnp.abs(d))
    h = jnp.where(d >= 0.0, 1.0 / (ed + 1.0), ed / (1.0 + ed))
    scratch[...] = jnp.full((528, 768), -jnp.inf, jnp.float32)
    scratch[pl.ds(8, _H), pl.ds(128, _W)] = h
    pool = jnp.full((_H, _W), -jnp.inf, jnp.float32)
    for di in (7, 8, 9):
        for dj in (127, 128, 129):
            pool = jnp.maximum(pool, scratch[pl.ds(di, _H), pl.ds(dj, _W)])
    keep = jnp.logical_and(h > _THRESH, h == pool)
    out_ref[0] = jnp.where(keep, h, 0.0)


def _mask_call(d):
    return pl.pallas_call(
        _mask_kernel,
        grid=(_B,),
        in_specs=[
            pl.BlockSpec((1, _H, _W), lambda b: (b, 0, 0)),
        ],
        out_specs=pl.BlockSpec((1, _H, _W), lambda b: (b, 0, 0)),
        out_shape=jax.ShapeDtypeStruct((_B, _H, _W), jnp.float32),
        scratch_shapes=[pltpu.VMEM((528, 768), jnp.float32)],
        interpret=False,
    )(d)


def _decode_kernel(rows_ref, idx_ref, out_ref):
    n = _B * _KPAD
    rows128 = rows_ref[...]
    idxf = idx_ref[...]
    sub = idxf - jnp.floor(idxf * 0.25) * 4.0
    rows = jnp.zeros((n, _C), jnp.float32)
    for m in range(4):
        rows = rows + jnp.where(sub == m, rows128[:, m * _C:(m + 1) * _C], 0.0)
    col = lax.broadcasted_iota(jnp.int32, (n, _C), 1)
    colf = col.astype(jnp.float32)
    angle = 2.0 * np.pi / _NBIN

    def sel(c):
        return jnp.sum(jnp.where(col == c, rows, 0.0), axis=1, keepdims=True)

    binm = jnp.logical_and(col >= 5, col <= 16)
    bmax = jnp.max(jnp.where(binm, rows, -jnp.inf), axis=1, keepdims=True)
    bidx = jnp.min(
        jnp.where(jnp.logical_and(binm, rows == bmax), colf - 5.0, 1e9),
        axis=1, keepdims=True)
    res = jnp.sum(
        jnp.where(jnp.logical_and(col >= 17, colf - 17.0 == bidx), rows, 0.0),
        axis=1, keepdims=True)
    heading = jnp.mod(bidx * angle + res * (angle * 0.5), 2.0 * np.pi)
    heading = jnp.where(heading > np.pi, heading - 2.0 * np.pi, heading)

    sl = sel(29) * 4.7 + 4.7
    sw = sel(30) * 2.1 + 2.1
    sh = sel(31) * 1.7 + 1.7
    ix = jnp.floor(idxf / 512.0)
    iy = idxf - ix * 512.0
    cx = (-81.92 + (ix + 0.5) * 0.32) + sel(2)
    cy = (-81.92 + (iy + 0.5) * 0.32) + sel(3)
    cz = sel(4)

    ocol = lax.broadcasted_iota(jnp.int32, (n, 8), 1)
    out = jnp.zeros((n, 8), jnp.float32)
    for c, v in enumerate((cx, cy, cz, sl, sw, sh, heading)):
        out = out + jnp.where(ocol == c, v, 0.0)
    out_ref[...] = out


def _decode_call(rows, idxf):
    return pl.pallas_call(
        _decode_kernel,
        out_shape=jax.ShapeDtypeStruct((_B * _KPAD, 8), jnp.float32),
        interpret=False,
    )(rows, idxf)


def _sc_gather(table, gidx):
    info = plsc.get_sparse_core_info()
    nc, ns = info.num_cores, info.num_subcores
    nw = nc * ns
    bpw = (_B * _KPAD) // nw
    mesh = plsc.VectorSubcoreMesh(core_axis_name="c", subcore_axis_name="s")

    @functools.partial(
        pl.kernel, mesh=mesh,
        out_type=jax.ShapeDtypeStruct((_B * _KPAD, 128), jnp.float32),
        scratch_types=[
            pltpu.VMEM((bpw,), jnp.int32),
            pltpu.VMEM((bpw, 128), jnp.float32),
            pltpu.SemaphoreType.DMA,
        ],
    )
    def gk(table_hbm, idx_hbm, out_hbm, idx_v, rows_v, sem):
        wid = lax.axis_index("s") * nc + lax.axis_index("c")
        base = wid * bpw
        pltpu.sync_copy(idx_hbm.at[pl.ds(base, bpw)], idx_v)
        pltpu.async_copy(table_hbm.at[idx_v], rows_v, sem).wait()
        pltpu.sync_copy(rows_v, out_hbm.at[pl.ds(base, bpw)])

    return gk(table, gidx)


def kernel(prediction):
    d = prediction[..., 1] - prediction[..., 0]
    s = _mask_call(d)
    nc4 = 4
    v1, i1 = lax.top_k(s.reshape(_B * nc4, _HW // nc4), _K)
    gi = (i1.reshape(_B, nc4, _K)
          + (jnp.arange(nc4, dtype=jnp.int32) * (_HW // nc4))[None, :, None]
          ).reshape(_B, nc4 * _K)
    vals, pos = lax.top_k(v1.reshape(_B, nc4 * _K), _K)
    top_idx = jnp.take_along_axis(gi, pos, axis=1)
    pad_idx = jnp.pad(top_idx, ((0, 0), (0, _KPAD - _K)))
    gidx = (pad_idx
            + (jnp.arange(_B, dtype=jnp.int32) * _HW)[:, None]).reshape(-1)
    gsup = gidx // 4
    table = prediction.reshape((_B * _HW) // 4, _C * 4)
    rows = _sc_gather(table, gsup)
    idxf = pad_idx.astype(jnp.float32).reshape(_B * _KPAD, 1)
    boxes = _decode_call(rows, idxf)
    box_decoded = boxes.reshape(_B, _KPAD, 8)[:, :_K, :7]
    box_class = jnp.full((_B, _K), 1, jnp.int32)
    return box_decoded, box_class, vals


# speedup vs baseline: 2.1982x; 1.0476x over previous
import functools

import jax
import jax.numpy as jnp
import numpy as np
from jax import lax
from jax.experimental import pallas as pl
from jax.experimental.pallas import tpu as pltpu
from jax.experimental.pallas import tpu_sc as plsc

_B, _H, _W, _C = 4, 512, 512, 32
_HW = _H * _W
_K = 1000
_KPAD = 1024
_NBIN = 12
_THRESH = 0.2


def _mask_kernel(d_ref, out_ref, scratch):
    d = d_ref[0]
    ed = jnp.exp(-jnp.abs(d))
    h = jnp.where(d >= 0.0, 1.0 / (ed + 1.0), ed / (1.0 + ed))
    scratch[...] = jnp.full((528, 768), -jnp.inf, jnp.float32)
    scratch[pl.ds(8, _H), pl.ds(128, _W)] = h
    pool = jnp.full((_H, _W), -jnp.inf, jnp.float32)
    for di in (7, 8, 9):
        for dj in (127, 128, 129):
            pool = jnp.maximum(pool, scratch[pl.ds(di, _H), pl.ds(dj, _W)])
    keep = jnp.logical_and(h > _THRESH, h == pool)
    out_ref[0] = jnp.where(keep, h, 0.0)


def _mask_call(d):
    return pl.pallas_call(
        _mask_kernel,
        grid=(_B,),
        in_specs=[
            pl.BlockSpec((1, _H, _W), lambda b: (b, 0, 0)),
        ],
        out_specs=pl.BlockSpec((1, _H, _W), lambda b: (b, 0, 0)),
        out_shape=jax.ShapeDtypeStruct((_B, _H, _W), jnp.float32),
        scratch_shapes=[pltpu.VMEM((528, 768), jnp.float32)],
        interpret=False,
    )(d)


def _decode_kernel(rows_ref, idx_ref, out_ref):
    n = _B * _KPAD
    rows128 = rows_ref[...]
    idxf = idx_ref[...]
    sub = idxf - jnp.floor(idxf * 0.25) * 4.0
    rows = jnp.zeros((n, _C), jnp.float32)
    for m in range(4):
        rows = rows + jnp.where(sub == m, rows128[:, m * _C:(m + 1) * _C], 0.0)
    col = lax.broadcasted_iota(jnp.int32, (n, _C), 1)
    colf = col.astype(jnp.float32)
    angle = 2.0 * np.pi / _NBIN

    def sel(c):
        return jnp.sum(jnp.where(col == c, rows, 0.0), axis=1, keepdims=True)

    binm = jnp.logical_and(col >= 5, col <= 16)
    bmax = jnp.max(jnp.where(binm, rows, -jnp.inf), axis=1, keepdims=True)
    bidx = jnp.min(
        jnp.where(jnp.logical_and(binm, rows == bmax), colf - 5.0, 1e9),
        axis=1, keepdims=True)
    res = jnp.sum(
        jnp.where(jnp.logical_and(col >= 17, colf - 17.0 == bidx), rows, 0.0),
        axis=1, keepdims=True)
    heading = jnp.mod(bidx * angle + res * (angle * 0.5), 2.0 * np.pi)
    heading = jnp.where(heading > np.pi, heading - 2.0 * np.pi, heading)

    sl = sel(29) * 4.7 + 4.7
    sw = sel(30) * 2.1 + 2.1
    sh = sel(31) * 1.7 + 1.7
    ix = jnp.floor(idxf / 512.0)
    iy = idxf - ix * 512.0
    cx = (-81.92 + (ix + 0.5) * 0.32) + sel(2)
    cy = (-81.92 + (iy + 0.5) * 0.32) + sel(3)
    cz = sel(4)

    ocol = lax.broadcasted_iota(jnp.int32, (n, 8), 1)
    out = jnp.zeros((n, 8), jnp.float32)
    for c, v in enumerate((cx, cy, cz, sl, sw, sh, heading)):
        out = out + jnp.where(ocol == c, v, 0.0)
    out_ref[...] = out


def _decode_call(rows, idxf):
    return pl.pallas_call(
        _decode_kernel,
        out_shape=jax.ShapeDtypeStruct((_B * _KPAD, 8), jnp.float32),
        interpret=False,
    )(rows, idxf)


def _sc_gather(table, gidx):
    info = plsc.get_sparse_core_info()
    nc, ns = info.num_cores, info.num_subcores
    nw = nc * ns
    bpw = (_B * _KPAD) // nw
    mesh = plsc.VectorSubcoreMesh(core_axis_name="c", subcore_axis_name="s")

    @functools.partial(
        pl.kernel, mesh=mesh,
        out_type=jax.ShapeDtypeStruct((_B * _KPAD, 128), jnp.float32),
        scratch_types=[
            pltpu.VMEM((bpw,), jnp.int32),
            pltpu.VMEM((bpw, 128), jnp.float32),
            pltpu.SemaphoreType.DMA,
        ],
    )
    def gk(table_hbm, idx_hbm, out_hbm, idx_v, rows_v, sem):
        wid = lax.axis_index("s") * nc + lax.axis_index("c")
        base = wid * bpw
        pltpu.sync_copy(idx_hbm.at[pl.ds(base, bpw)], idx_v)
        pltpu.async_copy(table_hbm.at[idx_v], rows_v, sem).wait()
        pltpu.sync_copy(rows_v, out_hbm.at[pl.ds(base, bpw)])

    return gk(table, gidx)


def kernel(prediction):
    d = prediction[..., 1] - prediction[..., 0]
    s = _mask_call(d)
    nc4 = 8
    v1, i1 = lax.top_k(s.reshape(_B * nc4, _HW // nc4), _K)
    gi = (i1.reshape(_B, nc4, _K)
          + (jnp.arange(nc4, dtype=jnp.int32) * (_HW // nc4))[None, :, None]
          ).reshape(_B, nc4 * _K)
    vals, pos = lax.top_k(v1.reshape(_B, nc4 * _K), _K)
    top_idx = jnp.take_along_axis(gi, pos, axis=1)
    pad_idx = jnp.pad(top_idx, ((0, 0), (0, _KPAD - _K)))
    gidx = (pad_idx
            + (jnp.arange(_B, dtype=jnp.int32) * _HW)[:, None]).reshape(-1)
    gsup = gidx // 4
    table = prediction.reshape((_B * _HW) // 4, _C * 4)
    rows = _sc_gather(table, gsup)
    idxf = pad_idx.astype(jnp.float32).reshape(_B * _KPAD, 1)
    boxes = _decode_call(rows, idxf)
    box_decoded = boxes.reshape(_B, _KPAD, 8)[:, :_K, :7]
    box_class = jnp.full((_B, _K), 1, jnp.int32)
    return box_decoded, box_class, vals


# speedup vs baseline: 2.2244x; 1.0119x over previous
import functools

import jax
import jax.numpy as jnp
import numpy as np
from jax import lax
from jax.experimental import pallas as pl
from jax.experimental.pallas import tpu as pltpu
from jax.experimental.pallas import tpu_sc as plsc

_B, _H, _W, _C = 4, 512, 512, 32
_HW = _H * _W
_K = 1000
_KPAD = 1024
_NBIN = 12
_THRESH = 0.2


def _mask_kernel(d_ref, out_ref, scratch):
    d = d_ref[0]
    ed = jnp.exp(-jnp.abs(d))
    h = jnp.where(d >= 0.0, 1.0 / (ed + 1.0), ed / (1.0 + ed))
    scratch[...] = jnp.full((528, 768), -jnp.inf, jnp.float32)
    scratch[pl.ds(8, _H), pl.ds(128, _W)] = h
    pool = jnp.full((_H, _W), -jnp.inf, jnp.float32)
    for di in (7, 8, 9):
        for dj in (127, 128, 129):
            pool = jnp.maximum(pool, scratch[pl.ds(di, _H), pl.ds(dj, _W)])
    keep = jnp.logical_and(h > _THRESH, h == pool)
    out_ref[0] = jnp.where(keep, h, 0.0)


def _mask_call(d):
    return pl.pallas_call(
        _mask_kernel,
        grid=(_B,),
        in_specs=[
            pl.BlockSpec((1, _H, _W), lambda b: (b, 0, 0)),
        ],
        out_specs=pl.BlockSpec((1, _H, _W), lambda b: (b, 0, 0)),
        out_shape=jax.ShapeDtypeStruct((_B, _H, _W), jnp.float32),
        scratch_shapes=[pltpu.VMEM((528, 768), jnp.float32)],
        interpret=False,
    )(d)


def _decode_kernel(rows_ref, idx_ref, out_ref):
    n = _B * _KPAD
    rows128 = rows_ref[...]
    idxf = idx_ref[...]
    sub = idxf - jnp.floor(idxf * 0.25) * 4.0
    rows = jnp.zeros((n, _C), jnp.float32)
    for m in range(4):
        rows = rows + jnp.where(sub == m, rows128[:, m * _C:(m + 1) * _C], 0.0)
    col = lax.broadcasted_iota(jnp.int32, (n, _C), 1)
    colf = col.astype(jnp.float32)
    angle = 2.0 * np.pi / _NBIN

    def sel(c):
        return jnp.sum(jnp.where(col == c, rows, 0.0), axis=1, keepdims=True)

    binm = jnp.logical_and(col >= 5, col <= 16)
    bmax = jnp.max(jnp.where(binm, rows, -jnp.inf), axis=1, keepdims=True)
    bidx = jnp.min(
        jnp.where(jnp.logical_and(binm, rows == bmax), colf - 5.0, 1e9),
        axis=1, keepdims=True)
    res = jnp.sum(
        jnp.where(jnp.logical_and(col >= 17, colf - 17.0 == bidx), rows, 0.0),
        axis=1, keepdims=True)
    heading = jnp.mod(bidx * angle + res * (angle * 0.5), 2.0 * np.pi)
    heading = jnp.where(heading > np.pi, heading - 2.0 * np.pi, heading)

    sl = sel(29) * 4.7 + 4.7
    sw = sel(30) * 2.1 + 2.1
    sh = sel(31) * 1.7 + 1.7
    ix = jnp.floor(idxf / 512.0)
    iy = idxf - ix * 512.0
    cx = (-81.92 + (ix + 0.5) * 0.32) + sel(2)
    cy = (-81.92 + (iy + 0.5) * 0.32) + sel(3)
    cz = sel(4)

    ocol = lax.broadcasted_iota(jnp.int32, (n, 8), 1)
    out = jnp.zeros((n, 8), jnp.float32)
    for c, v in enumerate((cx, cy, cz, sl, sw, sh, heading)):
        out = out + jnp.where(ocol == c, v, 0.0)
    out_ref[...] = out


def _decode_call(rows, idxf):
    return pl.pallas_call(
        _decode_kernel,
        out_shape=jax.ShapeDtypeStruct((_B * _KPAD, 8), jnp.float32),
        interpret=False,
    )(rows, idxf)


def _sc_gather(table, gidx):
    info = plsc.get_sparse_core_info()
    nc, ns = info.num_cores, info.num_subcores
    nw = nc * ns
    bpw = (_B * _KPAD) // nw
    mesh = plsc.VectorSubcoreMesh(core_axis_name="c", subcore_axis_name="s")

    @functools.partial(
        pl.kernel, mesh=mesh,
        out_type=jax.ShapeDtypeStruct((_B * _KPAD, 128), jnp.float32),
        scratch_types=[
            pltpu.VMEM((bpw,), jnp.int32),
            pltpu.VMEM((bpw, 128), jnp.float32),
            pltpu.SemaphoreType.DMA,
        ],
    )
    def gk(table_hbm, idx_hbm, out_hbm, idx_v, rows_v, sem):
        wid = lax.axis_index("s") * nc + lax.axis_index("c")
        base = wid * bpw
        pltpu.sync_copy(idx_hbm.at[pl.ds(base, bpw)], idx_v)
        pltpu.async_copy(table_hbm.at[idx_v], rows_v, sem).wait()
        pltpu.sync_copy(rows_v, out_hbm.at[pl.ds(base, bpw)])

    return gk(table, gidx)


def kernel(prediction):
    d = prediction[..., 1] - prediction[..., 0]
    s = _mask_call(d)
    nc4 = 16
    v1, i1 = lax.top_k(s.reshape(_B * nc4, _HW // nc4), _K)
    gi = (i1.reshape(_B, nc4, _K)
          + (jnp.arange(nc4, dtype=jnp.int32) * (_HW // nc4))[None, :, None]
          ).reshape(_B, nc4 * _K)
    vals, pos = lax.top_k(v1.reshape(_B, nc4 * _K), _K)
    top_idx = jnp.take_along_axis(gi, pos, axis=1)
    pad_idx = jnp.pad(top_idx, ((0, 0), (0, _KPAD - _K)))
    gidx = (pad_idx
            + (jnp.arange(_B, dtype=jnp.int32) * _HW)[:, None]).reshape(-1)
    gsup = gidx // 4
    table = prediction.reshape((_B * _HW) // 4, _C * 4)
    rows = _sc_gather(table, gsup)
    idxf = pad_idx.astype(jnp.float32).reshape(_B * _KPAD, 1)
    boxes = _decode_call(rows, idxf)
    box_decoded = boxes.reshape(_B, _KPAD, 8)[:, :_K, :7]
    box_class = jnp.full((_B, _K), 1, jnp.int32)
    return box_decoded, box_class, vals
